# speedup vs baseline: 5.8884x; 5.8884x over previous
import functools

import jax
import jax.numpy as jnp
from jax import lax
from jax.experimental import pallas as pl
from jax.experimental.pallas import tpu as pltpu
from jax.experimental.pallas import tpu_sc as plsc

_N0 = 5000
_N = 10000
_NP = 10240
_E = 320000
_D = 128

_B = 128
_NTILES = 32
_NCHUNK = 79
_EPT = _NCHUNK * _B
_EP = _NTILES * _EPT
_EPR = _EP // 128
_RPT = _NP // 16
_RB = 128


def _mesh():
    return plsc.VectorSubcoreMesh(core_axis_name="c", subcore_axis_name="s")


def _sc_deg_body(srcd, dsts, ones_h, zeros_h, out_od, out_id,
                 si_v, di_v, ones_v, od_s, id_s, sem):
    c = lax.axis_index("c")
    s = lax.axis_index("s")
    tid = c * 16 + s
    pltpu.sync_copy(srcd.at[tid], si_v)
    pltpu.sync_copy(dsts.at[tid], di_v)
    pltpu.sync_copy(ones_h, ones_v)
    pltpu.sync_copy(zeros_h, od_s.at[pl.ds(s * _RPT, _RPT)])
    pltpu.sync_copy(zeros_h, id_s.at[pl.ds(s * _RPT, _RPT)])
    plsc.subcore_barrier()

    def step(j, carry):
        pltpu.sync_copy(ones_v, od_s.at[si_v.at[j]], add=True)
        pltpu.sync_copy(ones_v, id_s.at[di_v.at[j]], add=True)
        return carry

    lax.fori_loop(0, _NCHUNK, step, 0)
    plsc.subcore_barrier()
    sl = pl.ds(s * _RPT, _RPT)
    pltpu.sync_copy(od_s.at[sl], out_od.at[c].at[sl])
    pltpu.sync_copy(id_s.at[sl], out_id.at[c].at[sl])


_sc_deg = functools.partial(
    pl.kernel,
    out_type=(
        jax.ShapeDtypeStruct((2, _NP, 16), jnp.float32),
        jax.ShapeDtypeStruct((2, _NP, 16), jnp.float32),
    ),
    mesh=_mesh(),
    scratch_types=[
        pltpu.VMEM((_NCHUNK, _B), jnp.int32),
        pltpu.VMEM((_NCHUNK, _B), jnp.int32),
        pltpu.VMEM((_B, 16), jnp.float32),
        pltpu.VMEM_SHARED((_NP, 16), jnp.float32),
        pltpu.VMEM_SHARED((_NP, 16), jnp.float32),
        pltpu.SemaphoreType.DMA,
    ],
)(_sc_deg_body)


def _sc_agg_body(hs, srcg, dsts, zeros_h, out,
                 si_v, di_v, rows_v, agg_s, sem):
    c = lax.axis_index("c")
    s = lax.axis_index("s")
    tid = c * 16 + s
    pltpu.sync_copy(srcg.at[tid], si_v)
    pltpu.sync_copy(dsts.at[tid], di_v)
    pltpu.sync_copy(zeros_h, agg_s.at[pl.ds(s * _RPT, _RPT)])
    plsc.subcore_barrier()

    def step(j, carry):
        pltpu.async_copy(hs.at[si_v.at[j]], rows_v, sem).wait()
        pltpu.sync_copy(rows_v, agg_s.at[di_v.at[j]], add=True)
        return carry

    lax.fori_loop(0, _NCHUNK, step, 0)
    plsc.subcore_barrier()
    sl = pl.ds(s * _RPT, _RPT)
    pltpu.sync_copy(agg_s.at[sl], out.at[c].at[sl])


_sc_agg = functools.partial(
    pl.kernel,
    out_type=jax.ShapeDtypeStruct((2, _NP, _D), jnp.float32),
    mesh=_mesh(),
    scratch_types=[
        pltpu.VMEM((_NCHUNK, _B), jnp.int32),
        pltpu.VMEM((_NCHUNK, _B), jnp.int32),
        pltpu.VMEM((_B, _D), jnp.float32),
        pltpu.VMEM_SHARED((_NP, _D), jnp.float32),
        pltpu.SemaphoreType.DMA,
    ],
)(_sc_agg_body)


def _prep_body(src_ref, dst_ref, srcd_ref, dsts_ref):
    srcv = src_ref[...]
    dstv = dst_ref[...]
    keep = srcv != dstv
    srcd_ref[...] = jnp.where(keep, srcv, _N)
    dsts_ref[...] = jnp.where(keep, dstv, _N)


_prep = pl.pallas_call(
    _prep_body,
    out_shape=(
        jax.ShapeDtypeStruct((_EPR, 128), jnp.int32),
        jax.ShapeDtypeStruct((_EPR, 128), jnp.int32),
    ),
)


def _norm_cols(degp_ref):
    od = degp_ref[0, 0][:, 0:1] + degp_ref[1, 0][:, 0:1] + 1.0
    idg = degp_ref[0, 1][:, 0:1] + degp_ref[1, 1][:, 0:1] + 1.0
    return lax.rsqrt(od), lax.rsqrt(idg)


def _proj_body(feat_ref, w0_ref, b0_ref, w1_ref, b1_ref, degp_ref, hs1_ref):
    i = pl.program_id(0)
    feat = feat_ref[...]
    h0 = jnp.dot(feat, w0_ref[...], preferred_element_type=jnp.float32) + b0_ref[...]
    h1 = jnp.dot(feat, w1_ref[...], preferred_element_type=jnp.float32) + b1_ref[...]
    rows = i * _RB + lax.broadcasted_iota(jnp.int32, (_RB, 1), 0)
    h = jnp.where(rows < _N0, h0, h1)
    ns, _ = _norm_cols(degp_ref)
    hs1_ref[...] = h * ns


_proj = pl.pallas_call(
    _proj_body,
    grid=(_NP // _RB,),
    in_specs=[
        pl.BlockSpec((_RB, _D), lambda i: (i, 0)),
        pl.BlockSpec((_D, _D), lambda i: (0, 0)),
        pl.BlockSpec((1, _D), lambda i: (0, 0)),
        pl.BlockSpec((_D, _D), lambda i: (0, 0)),
        pl.BlockSpec((1, _D), lambda i: (0, 0)),
        pl.BlockSpec((2, 2, _RB, 16), lambda i: (0, 0, i, 0)),
    ],
    out_specs=pl.BlockSpec((_RB, _D), lambda i: (i, 0)),
    out_shape=jax.ShapeDtypeStruct((_NP, _D), jnp.float32),
)


def _mid_body(p_ref, hs1_ref, degp_ref, bg1_ref, hs2_ref):
    agg = p_ref[0] + p_ref[1] + hs1_ref[...]
    ns, nd = _norm_cols(degp_ref)
    h1 = jnp.maximum(agg * nd + bg1_ref[...], 0.0)
    hs2_ref[...] = h1 * ns


_mid = pl.pallas_call(
    _mid_body,
    grid=(_NP // _RB,),
    in_specs=[
        pl.BlockSpec((2, _RB, _D), lambda i: (0, i, 0)),
        pl.BlockSpec((_RB, _D), lambda i: (i, 0)),
        pl.BlockSpec((2, 2, _RB, 16), lambda i: (0, 0, i, 0)),
        pl.BlockSpec((1, _D), lambda i: (0, 0)),
    ],
    out_specs=pl.BlockSpec((_RB, _D), lambda i: (i, 0)),
    out_shape=jax.ShapeDtypeStruct((_NP, _D), jnp.float32),
)


def _fin_body(p_ref, hs2_ref, degp_ref, wg_ref, bg2_ref, out_ref):
    agg = p_ref[0] + p_ref[1] + hs2_ref[...]
    _, nd = _norm_cols(degp_ref)
    rst = agg * nd
    out_ref[...] = jnp.maximum(
        jnp.dot(rst, wg_ref[...], preferred_element_type=jnp.float32)
        + bg2_ref[...], 0.0)


_fin = pl.pallas_call(
    _fin_body,
    grid=(_NP // _RB,),
    in_specs=[
        pl.BlockSpec((2, _RB, _D), lambda i: (0, i, 0)),
        pl.BlockSpec((_RB, _D), lambda i: (i, 0)),
        pl.BlockSpec((2, 2, _RB, 16), lambda i: (0, 0, i, 0)),
        pl.BlockSpec((_D, _D), lambda i: (0, 0)),
        pl.BlockSpec((1, _D), lambda i: (0, 0)),
    ],
    out_specs=pl.BlockSpec((_RB, _D), lambda i: (i, 0)),
    out_shape=jax.ShapeDtypeStruct((_NP, _D), jnp.float32),
)


def kernel(feat_type0, feat_type1, edge_index, W0, b0, W1, b1, b_g1, W_g2, b_g2):
    src = edge_index[0]
    dst = edge_index[1]
    pad = _EP - _E
    srcp = jnp.pad(src, (0, pad), constant_values=_N).reshape(_EPR, 128)
    dstp = jnp.pad(dst, (0, pad), constant_values=_N).reshape(_EPR, 128)
    srcd, dsts = _prep(srcp, dstp)
    srcg3 = srcp.reshape(_NTILES, _NCHUNK, _B)
    srcd3 = srcd.reshape(_NTILES, _NCHUNK, _B)
    dsts3 = dsts.reshape(_NTILES, _NCHUNK, _B)

    ones16 = jnp.ones((_B, 16), jnp.float32)
    zeros16 = jnp.zeros((_RPT, 16), jnp.float32)
    zeros128 = jnp.zeros((_RPT, _D), jnp.float32)

    od_parts, id_parts = _sc_deg(srcd3, dsts3, ones16, zeros16)
    degp = jnp.stack([od_parts, id_parts], axis=1)

    feat = jnp.concatenate([feat_type0, feat_type1], axis=0)
    feat = jnp.pad(feat, ((0, _NP - _N), (0, 0)))

    hs1 = _proj(feat, W0, b0.reshape(1, _D), W1, b1.reshape(1, _D), degp)
    parts1 = _sc_agg(hs1, srcg3, dsts3, zeros128)
    hs2 = _mid(parts1, hs1, degp, b_g1.reshape(1, _D))
    parts2 = _sc_agg(hs2, srcg3, dsts3, zeros128)
    out = _fin(parts2, hs2, degp, W_g2, b_g2.reshape(1, _D))
    return out[:_N0], out[_N0:_N]
